# baseline (speedup 1.0000x reference)
import jax
import jax.numpy as jnp
from jax import lax
from jax.experimental import pallas as pl
from jax.experimental.pallas import tpu as pltpu
from jax.experimental.pallas import tpu_sc as plsc

_B, _T, _D, _E, _H = 1, 2048, 768, 8, 1536
_N = _B * _T
_TILE = 128
_NCHUNK = _N // _TILE
_MAXTILES = 2 * _N // _TILE + _E
_PADDED = _MAXTILES * _TILE


def _route_kernel(x_ref, gw_ref, d1_ref, d2_ref, w0_ref, w1_ref,
                  te_ref, loss_ref):
    x = x_ref[...]
    gw = gw_ref[...]
    logits = lax.dot_general(
        x, gw, (((1,), (1,)), ((), ())),
        preferred_element_type=jnp.float32)
    p = jax.nn.softmax(logits, axis=-1)
    rank = jnp.zeros_like(p)
    eidx = lax.broadcasted_iota(jnp.int32, (1, _E), 1)
    for j in range(_E):
        pj = p[:, j:j + 1]
        rank += (pj > p).astype(p.dtype)
        rank += ((pj == p) & (j < eidx)).astype(p.dtype)
    sel = (rank < 2.0).astype(p.dtype)
    w = p * sel
    wd = w / jnp.sum(w, axis=1, keepdims=True)
    ii = lax.broadcasted_iota(jnp.int32, (_TILE, _TILE), 0)
    jj = lax.broadcasted_iota(jnp.int32, (_TILE, _TILE), 1)
    tril = (ii >= jj).astype(jnp.float32)
    carry = jnp.zeros((1, _E), jnp.float32)
    cum_rows = []
    for c in range(_NCHUNK):
        sc = lax.slice(sel, (c * _TILE, 0), ((c + 1) * _TILE, _E))
        cum_rows.append(
            lax.dot_general(tril, sc, (((1,), (0,)), ((), ())),
                            preferred_element_type=jnp.float32) + carry)
        carry = carry + jnp.sum(sc, axis=0, keepdims=True)
    cum = jnp.concatenate(cum_rows, axis=0)
    cnt = carry
    pmean = jnp.mean(p, axis=0, keepdims=True)
    loss_ref[...] = (jnp.sum(pmean * cnt) * (_E / _N)).reshape(1, 1)
    ptiles = jnp.floor((cnt + (_TILE - 1.0)) * (1.0 / _TILE))
    ltmask = (lax.broadcasted_iota(jnp.int32, (_E, _E), 0) <
              lax.broadcasted_iota(jnp.int32, (_E, _E), 1)).astype(jnp.float32)
    tstart = lax.dot_general(ptiles, ltmask, (((1,), (0,)), ((), ())),
                             preferred_element_type=jnp.float32)
    pstart = _TILE * tstart
    kcnt = lax.dot_general(sel, ltmask, (((1,), (0,)), ((), ())),
                           preferred_element_type=jnp.float32)
    pos = pstart + cum - 1.0
    is0 = sel * (kcnt == 0.0)
    is1 = sel * (kcnt == 1.0)
    d1 = jnp.sum(pos * is0, axis=1, keepdims=True)
    d2 = jnp.sum(pos * is1, axis=1, keepdims=True)
    d1_ref[...] = jnp.concatenate([2.0 * d1, 2.0 * d1 + 1.0],
                                  axis=1).astype(jnp.int32)
    d2_ref[...] = jnp.concatenate([2.0 * d2, 2.0 * d2 + 1.0],
                                  axis=1).astype(jnp.int32)
    w0_ref[...] = jnp.sum(wd * is0, axis=1, keepdims=True)
    w1_ref[...] = jnp.sum(wd * is1, axis=1, keepdims=True)
    tend = tstart + ptiles
    tio = lax.broadcasted_iota(jnp.int32, (_MAXTILES, _E), 0).astype(jnp.float32)
    te = jnp.sum((tend <= tio).astype(jnp.float32), axis=1, keepdims=True)
    te_ref[...] = jnp.minimum(te, float(_E - 1)).astype(jnp.int32)


def _route(x_flat, gate_w):
    return pl.pallas_call(
        _route_kernel,
        out_shape=[
            jax.ShapeDtypeStruct((_N, 2), jnp.int32),
            jax.ShapeDtypeStruct((_N, 2), jnp.int32),
            jax.ShapeDtypeStruct((_N, 1), jnp.float32),
            jax.ShapeDtypeStruct((_N, 1), jnp.float32),
            jax.ShapeDtypeStruct((_MAXTILES, 1), jnp.int32),
            jax.ShapeDtypeStruct((1, 1), jnp.float32),
        ],
    )(x_flat, gate_w)


_HD = _D // 2
_SCWIN = 128


def _sc_scatter(x_half, dests):
    @pl.kernel(
        out_type=jax.ShapeDtypeStruct((2 * _PADDED, _HD), jnp.float32),
        mesh=plsc.VectorSubcoreMesh(core_axis_name="core",
                                    subcore_axis_name="subcore"))
    def k(x_hbm, i_hbm, o_hbm):
        def body(x_vmem, i_vmem):
            pltpu.sync_copy(x_vmem, o_hbm.at[i_vmem.at[0]])

        pltpu.emit_pipeline(
            body,
            grid=(4 * _N // _SCWIN,),
            in_specs=[
                pl.BlockSpec((_SCWIN, _HD),
                             lambda i: (lax.rem(i, 2 * _N // _SCWIN), 0)),
                pl.BlockSpec((1, _SCWIN), lambda i: (0, i)),
            ],
            out_specs=[],
            core_axis_name=("core", "subcore"),
            dimension_semantics=(pltpu.PARALLEL,),
        )(x_hbm, i_hbm)

    return k(x_half, dests)


def _sc_gather(ys_half, dests):
    @pl.kernel(
        out_type=jax.ShapeDtypeStruct((4 * _N, _HD), jnp.float32),
        mesh=plsc.VectorSubcoreMesh(core_axis_name="core",
                                    subcore_axis_name="subcore"))
    def k(y_hbm, i_hbm, o_hbm):
        def body(i_vmem, o_vmem):
            pltpu.sync_copy(y_hbm.at[i_vmem.at[0]], o_vmem)

        pltpu.emit_pipeline(
            body,
            grid=(4 * _N // _SCWIN,),
            in_specs=[pl.BlockSpec((1, _SCWIN), lambda i: (0, i))],
            out_specs=[pl.BlockSpec((_SCWIN, _HD), lambda i: (i, 0))],
            core_axis_name=("core", "subcore"),
            dimension_semantics=(pltpu.PARALLEL,),
        )(i_hbm, o_hbm)

    return k(ys_half, dests)


def _expert_kernel(te_ref, xs_ref, wfc_ref, wproj_ref, ys_ref):
    xsb = xs_ref[...].astype(jnp.bfloat16)
    wfc = wfc_ref[0].astype(jnp.bfloat16)
    h = lax.dot_general(xsb, wfc, (((1,), (1,)), ((), ())),
                        preferred_element_type=jnp.float32)
    h = jnp.square(jnp.maximum(h, 0.0)).astype(jnp.bfloat16)
    wp = wproj_ref[0].astype(jnp.bfloat16)
    ys_ref[...] = lax.dot_general(h, wp, (((1,), (1,)), ((), ())),
                                  preferred_element_type=jnp.float32)


def _grouped_matmul(te, xs, w_fc, w_proj):
    grid_spec = pltpu.PrefetchScalarGridSpec(
        num_scalar_prefetch=1,
        grid=(_MAXTILES,),
        in_specs=[
            pl.BlockSpec((_TILE, _D), lambda t, te_ref: (t, 0)),
            pl.BlockSpec((1, _H, _D), lambda t, te_ref: (te_ref[t], 0, 0)),
            pl.BlockSpec((1, _D, _H), lambda t, te_ref: (te_ref[t], 0, 0)),
        ],
        out_specs=pl.BlockSpec((_TILE, _D), lambda t, te_ref: (t, 0)),
    )
    return pl.pallas_call(
        _expert_kernel,
        grid_spec=grid_spec,
        out_shape=jax.ShapeDtypeStruct((_PADDED, _D), jnp.float32),
    )(te, xs, w_fc, w_proj)


def _combine_kernel(g_ref, w0_ref, w1_ref, out_ref):
    g0 = g_ref[0:_N, :]
    g1 = g_ref[_N:2 * _N, :]
    out_ref[...] = w0_ref[...] * g0 + w1_ref[...] * g1


def _combine(g01, w0, w1):
    return pl.pallas_call(
        _combine_kernel,
        out_shape=jax.ShapeDtypeStruct((_N, _D), jnp.float32),
    )(g01, w0, w1)


def kernel(x, gate_w, w_fc, w_proj):
    x_flat = x.reshape(_N, _D)
    d1, d2, w0, w1, te, loss = _route(x_flat, gate_w)
    dests = jnp.concatenate([d1.reshape(1, 2 * _N), d2.reshape(1, 2 * _N)],
                            axis=1)
    xs_half = _sc_scatter(x_flat.reshape(2 * _N, _HD), dests)
    ys = _grouped_matmul(te.reshape(-1), xs_half.reshape(_PADDED, _D),
                         w_fc, w_proj)
    g_half = _sc_gather(ys.reshape(2 * _PADDED, _HD), dests)
    out = _combine(g_half.reshape(2 * _N, _D), w0, w1)
    return out.reshape(_B, _T, _D), loss[0, 0]


# speedup vs baseline: 1.1265x; 1.1265x over previous
import jax
import jax.numpy as jnp
from jax import lax
from jax.experimental import pallas as pl
from jax.experimental.pallas import tpu as pltpu
from jax.experimental.pallas import tpu_sc as plsc

_B, _T, _D, _E, _H = 1, 2048, 768, 8, 1536
_N = _B * _T
_TILE = 128
_NCHUNK = _N // _TILE
_GTILE = 256
_MAXTILES = 2 * _N // _GTILE + _E
_PADDED = _MAXTILES * _GTILE


def _route_kernel(x_ref, gw_ref, d1_ref, d2_ref, w0_ref, w1_ref,
                  te_ref, loss_ref):
    x = x_ref[...]
    gw = gw_ref[...]
    logits = lax.dot_general(
        x, gw, (((1,), (1,)), ((), ())),
        preferred_element_type=jnp.float32)
    p = jax.nn.softmax(logits, axis=-1)
    rank = jnp.zeros_like(p)
    eidx = lax.broadcasted_iota(jnp.int32, (1, _E), 1)
    for j in range(_E):
        pj = p[:, j:j + 1]
        rank += (pj > p).astype(p.dtype)
        rank += ((pj == p) & (j < eidx)).astype(p.dtype)
    sel = (rank < 2.0).astype(p.dtype)
    w = p * sel
    wd = w / jnp.sum(w, axis=1, keepdims=True)
    ii = lax.broadcasted_iota(jnp.int32, (_TILE, _TILE), 0)
    jj = lax.broadcasted_iota(jnp.int32, (_TILE, _TILE), 1)
    tril = (ii >= jj).astype(jnp.float32)
    carry = jnp.zeros((1, _E), jnp.float32)
    cum_rows = []
    for c in range(_NCHUNK):
        sc = lax.slice(sel, (c * _TILE, 0), ((c + 1) * _TILE, _E))
        cum_rows.append(
            lax.dot_general(tril, sc, (((1,), (0,)), ((), ())),
                            preferred_element_type=jnp.float32) + carry)
        carry = carry + jnp.sum(sc, axis=0, keepdims=True)
    cum = jnp.concatenate(cum_rows, axis=0)
    cnt = carry
    pmean = jnp.mean(p, axis=0, keepdims=True)
    loss_ref[...] = (jnp.sum(pmean * cnt) * (_E / _N)).reshape(1, 1)
    ptiles = jnp.floor((cnt + (_GTILE - 1.0)) * (1.0 / _GTILE))
    ltmask = (lax.broadcasted_iota(jnp.int32, (_E, _E), 0) <
              lax.broadcasted_iota(jnp.int32, (_E, _E), 1)).astype(jnp.float32)
    tstart = lax.dot_general(ptiles, ltmask, (((1,), (0,)), ((), ())),
                             preferred_element_type=jnp.float32)
    pstart = _GTILE * tstart
    kcnt = lax.dot_general(sel, ltmask, (((1,), (0,)), ((), ())),
                           preferred_element_type=jnp.float32)
    pos = pstart + cum - 1.0
    is0 = sel * (kcnt == 0.0)
    is1 = sel * (kcnt == 1.0)
    d1 = jnp.sum(pos * is0, axis=1, keepdims=True)
    d2 = jnp.sum(pos * is1, axis=1, keepdims=True)
    d1_ref[...] = jnp.concatenate([2.0 * d1, 2.0 * d1 + 1.0],
                                  axis=1).astype(jnp.int32)
    d2_ref[...] = jnp.concatenate([2.0 * d2, 2.0 * d2 + 1.0],
                                  axis=1).astype(jnp.int32)
    w0_ref[...] = jnp.sum(wd * is0, axis=1, keepdims=True)
    w1_ref[...] = jnp.sum(wd * is1, axis=1, keepdims=True)
    tend = tstart + ptiles
    tio = lax.broadcasted_iota(jnp.int32, (_MAXTILES, _E), 0).astype(jnp.float32)
    te = jnp.sum((tend <= tio).astype(jnp.float32), axis=1, keepdims=True)
    te_ref[...] = jnp.minimum(te, float(_E - 1)).astype(jnp.int32)


def _route(x_flat, gate_w):
    return pl.pallas_call(
        _route_kernel,
        out_shape=[
            jax.ShapeDtypeStruct((_N, 2), jnp.int32),
            jax.ShapeDtypeStruct((_N, 2), jnp.int32),
            jax.ShapeDtypeStruct((_N, 1), jnp.float32),
            jax.ShapeDtypeStruct((_N, 1), jnp.float32),
            jax.ShapeDtypeStruct((_MAXTILES, 1), jnp.int32),
            jax.ShapeDtypeStruct((1, 1), jnp.float32),
        ],
    )(x_flat, gate_w)


_HD = _D // 2
_SCWIN = 128


def _sc_scatter(x_half, dests):
    @pl.kernel(
        out_type=jax.ShapeDtypeStruct((2 * _PADDED, _HD), jnp.float32),
        mesh=plsc.VectorSubcoreMesh(core_axis_name="core",
                                    subcore_axis_name="subcore"))
    def k(x_hbm, i_hbm, o_hbm):
        def body(x_vmem, i_vmem):
            pltpu.sync_copy(x_vmem, o_hbm.at[i_vmem.at[0]])

        pltpu.emit_pipeline(
            body,
            grid=(4 * _N // _SCWIN,),
            in_specs=[
                pl.BlockSpec((_SCWIN, _HD),
                             lambda i: (lax.rem(i, 2 * _N // _SCWIN), 0)),
                pl.BlockSpec((1, _SCWIN), lambda i: (0, i)),
            ],
            out_specs=[],
            core_axis_name=("core", "subcore"),
            dimension_semantics=(pltpu.PARALLEL,),
        )(x_hbm, i_hbm)

    return k(x_half, dests)


def _sc_gather(ys_half, dests):
    @pl.kernel(
        out_type=jax.ShapeDtypeStruct((4 * _N, _HD), jnp.float32),
        mesh=plsc.VectorSubcoreMesh(core_axis_name="core",
                                    subcore_axis_name="subcore"))
    def k(y_hbm, i_hbm, o_hbm):
        def body(i_vmem, o_vmem):
            pltpu.sync_copy(y_hbm.at[i_vmem.at[0]], o_vmem)

        pltpu.emit_pipeline(
            body,
            grid=(4 * _N // _SCWIN,),
            in_specs=[pl.BlockSpec((1, _SCWIN), lambda i: (0, i))],
            out_specs=[pl.BlockSpec((_SCWIN, _HD), lambda i: (i, 0))],
            core_axis_name=("core", "subcore"),
            dimension_semantics=(pltpu.PARALLEL,),
        )(i_hbm, o_hbm)

    return k(ys_half, dests)


def _expert_kernel(te_ref, xs_ref, wfc_ref, wproj_ref, ys_ref):
    xsb = xs_ref[...].astype(jnp.bfloat16)
    wfc = wfc_ref[0].astype(jnp.bfloat16)
    h = lax.dot_general(xsb, wfc, (((1,), (1,)), ((), ())),
                        preferred_element_type=jnp.float32)
    h = jnp.square(jnp.maximum(h, 0.0)).astype(jnp.bfloat16)
    wp = wproj_ref[0].astype(jnp.bfloat16)
    ys_ref[...] = lax.dot_general(h, wp, (((1,), (1,)), ((), ())),
                                  preferred_element_type=jnp.float32)


def _grouped_matmul(te, xs, w_fc, w_proj):
    grid_spec = pltpu.PrefetchScalarGridSpec(
        num_scalar_prefetch=1,
        grid=(_MAXTILES,),
        in_specs=[
            pl.BlockSpec((_GTILE, _D), lambda t, te_ref: (t, 0)),
            pl.BlockSpec((1, _H, _D), lambda t, te_ref: (te_ref[t], 0, 0)),
            pl.BlockSpec((1, _D, _H), lambda t, te_ref: (te_ref[t], 0, 0)),
        ],
        out_specs=pl.BlockSpec((_GTILE, _D), lambda t, te_ref: (t, 0)),
    )
    return pl.pallas_call(
        _expert_kernel,
        grid_spec=grid_spec,
        out_shape=jax.ShapeDtypeStruct((_PADDED, _D), jnp.float32),
    )(te, xs, w_fc, w_proj)


def _combine_kernel(g_ref, w0_ref, w1_ref, out_ref):
    g0 = g_ref[0:_N, :]
    g1 = g_ref[_N:2 * _N, :]
    out_ref[...] = w0_ref[...] * g0 + w1_ref[...] * g1


def _combine(g01, w0, w1):
    return pl.pallas_call(
        _combine_kernel,
        out_shape=jax.ShapeDtypeStruct((_N, _D), jnp.float32),
    )(g01, w0, w1)


def kernel(x, gate_w, w_fc, w_proj):
    x_flat = x.reshape(_N, _D)
    d1, d2, w0, w1, te, loss = _route(x_flat, gate_w)
    dests = jnp.concatenate([d1.reshape(1, 2 * _N), d2.reshape(1, 2 * _N)],
                            axis=1)
    xs_half = _sc_scatter(x_flat.reshape(2 * _N, _HD), dests)
    ys = _grouped_matmul(te.reshape(-1), xs_half.reshape(_PADDED, _D),
                         w_fc, w_proj)
    g_half = _sc_gather(ys.reshape(2 * _PADDED, _HD), dests)
    out = _combine(g_half.reshape(2 * _N, _D), w0, w1)
    return out.reshape(_B, _T, _D), loss[0, 0]


# speedup vs baseline: 1.5435x; 1.3702x over previous
import jax
import jax.numpy as jnp
from jax import lax
from jax.experimental import pallas as pl
from jax.experimental.pallas import tpu as pltpu
from jax.experimental.pallas import tpu_sc as plsc

_B, _T, _D, _E, _H = 1, 2048, 768, 8, 1536
_N = _B * _T
_HD = _D // 2
_TILE = 128
_NCHUNK = _N // _TILE
_GTILE = 256
_MAXTILES = 2 * _N // _GTILE + _E
_PADDED = _MAXTILES * _GTILE
_SCWIN = 128


def _route_kernel(x_ref, gw_ref, x2_ref, d1_ref, d2_ref, w0_ref, w1_ref,
                  es_ref, ec_ref, loss_ref):
    x = x_ref[...]
    x2_ref[0:_N, :] = x[:, 0:_HD]
    x2_ref[_N:2 * _N, :] = x[:, _HD:_D]
    gw = gw_ref[...]
    logits = lax.dot_general(
        x, gw, (((1,), (1,)), ((), ())),
        preferred_element_type=jnp.float32)
    p = jax.nn.softmax(logits, axis=-1)
    rank = jnp.zeros_like(p)
    eidx = lax.broadcasted_iota(jnp.int32, (1, _E), 1)
    for j in range(_E):
        pj = p[:, j:j + 1]
        rank += (pj > p).astype(p.dtype)
        rank += ((pj == p) & (j < eidx)).astype(p.dtype)
    sel = (rank < 2.0).astype(p.dtype)
    w = p * sel
    wd = w / jnp.sum(w, axis=1, keepdims=True)
    ii = lax.broadcasted_iota(jnp.int32, (_TILE, _TILE), 0)
    jj = lax.broadcasted_iota(jnp.int32, (_TILE, _TILE), 1)
    tril = (ii >= jj).astype(jnp.float32)
    carry = jnp.zeros((1, _E), jnp.float32)
    cum_rows = []
    for c in range(_NCHUNK):
        sc = lax.slice(sel, (c * _TILE, 0), ((c + 1) * _TILE, _E))
        cum_rows.append(
            lax.dot_general(tril, sc, (((1,), (0,)), ((), ())),
                            preferred_element_type=jnp.float32) + carry)
        carry = carry + jnp.sum(sc, axis=0, keepdims=True)
    cum = jnp.concatenate(cum_rows, axis=0)
    cnt = carry
    pmean = jnp.mean(p, axis=0, keepdims=True)
    loss_ref[...] = (jnp.sum(pmean * cnt) * (_E / _N)).reshape(1, 1)
    ptiles = jnp.floor((cnt + (_GTILE - 1.0)) * (1.0 / _GTILE))
    ltmask = (lax.broadcasted_iota(jnp.int32, (_E, _E), 0) <
              lax.broadcasted_iota(jnp.int32, (_E, _E), 1)).astype(jnp.float32)
    tstart = lax.dot_general(ptiles, ltmask, (((1,), (0,)), ((), ())),
                             preferred_element_type=jnp.float32)
    pstart = _GTILE * tstart
    kcnt = lax.dot_general(sel, ltmask, (((1,), (0,)), ((), ())),
                           preferred_element_type=jnp.float32)
    pos = pstart + cum - 1.0
    is0 = sel * (kcnt == 0.0)
    is1 = sel * (kcnt == 1.0)
    d1_ref[...] = jnp.sum(pos * is0, axis=1, keepdims=True).astype(jnp.int32)
    d2_ref[...] = jnp.sum(pos * is1, axis=1, keepdims=True).astype(jnp.int32)
    w0_ref[...] = jnp.sum(wd * is0, axis=1, keepdims=True)
    w1_ref[...] = jnp.sum(wd * is1, axis=1, keepdims=True)
    es_ref[...] = jnp.transpose(tstart).astype(jnp.int32)
    ec_ref[...] = jnp.transpose(ptiles).astype(jnp.int32)


def _route(x_flat, gate_w):
    return pl.pallas_call(
        _route_kernel,
        out_shape=[
            jax.ShapeDtypeStruct((2 * _N, _HD), jnp.float32),
            jax.ShapeDtypeStruct((_N, 1), jnp.int32),
            jax.ShapeDtypeStruct((_N, 1), jnp.int32),
            jax.ShapeDtypeStruct((_N, 1), jnp.float32),
            jax.ShapeDtypeStruct((_N, 1), jnp.float32),
            jax.ShapeDtypeStruct((_E, 1), jnp.int32),
            jax.ShapeDtypeStruct((_E, 1), jnp.int32),
            jax.ShapeDtypeStruct((1, 1), jnp.float32),
        ],
    )(x_flat, gate_w)


def _sc_scatter(x2, dests):
    @pl.kernel(
        out_type=jax.ShapeDtypeStruct((2 * _PADDED, _HD), jnp.float32),
        mesh=plsc.VectorSubcoreMesh(core_axis_name="core",
                                    subcore_axis_name="subcore"))
    def k(x_hbm, i_hbm, o_hbm):
        def body(x_vmem, i_vmem):
            pltpu.sync_copy(x_vmem, o_hbm.at[i_vmem.at[0]])

        pltpu.emit_pipeline(
            body,
            grid=(4 * _N // _SCWIN,),
            in_specs=[
                pl.BlockSpec((_SCWIN, _HD),
                             lambda i: (lax.rem(i, 2 * _N // _SCWIN), 0)),
                pl.BlockSpec((1, _SCWIN), lambda i: (0, i)),
            ],
            out_specs=[],
            core_axis_name=("core", "subcore"),
            dimension_semantics=(pltpu.PARALLEL,),
        )(x_hbm, i_hbm)

    return k(x2, dests)


def _sc_gather(ys, dests):
    @pl.kernel(
        out_type=jax.ShapeDtypeStruct((4 * _N, _HD), jnp.float32),
        mesh=plsc.VectorSubcoreMesh(core_axis_name="core",
                                    subcore_axis_name="subcore"))
    def k(y_hbm, i_hbm, o_hbm):
        def body(i_vmem, o_vmem):
            pltpu.sync_copy(y_hbm.at[i_vmem.at[0]], o_vmem)

        pltpu.emit_pipeline(
            body,
            grid=(4 * _N // _SCWIN,),
            in_specs=[pl.BlockSpec((1, _SCWIN), lambda i: (0, i))],
            out_specs=[pl.BlockSpec((_SCWIN, _HD), lambda i: (i, 0))],
            core_axis_name=("core", "subcore"),
            dimension_semantics=(pltpu.PARALLEL,),
        )(i_hbm, o_hbm)

    return k(ys, dests)


def _expert_kernel(es_ref, ec_ref, xs_ref, wfc_ref, wp_ref, ys_ref,
                   xbuf, ybuf, isem, osem):
    e = pl.program_id(0)
    s0 = es_ref[e, 0]
    n = ec_ref[e, 0]
    wfc = wfc_ref[0].astype(jnp.bfloat16)
    wp = wp_ref[0].astype(jnp.bfloat16)

    def in_copies(i, slot):
        r = (s0 + i) * _GTILE
        return (
            pltpu.make_async_copy(
                xs_ref.at[pl.ds(r, _GTILE), pl.ds(0, _HD)],
                xbuf.at[slot, pl.ds(0, _GTILE), pl.ds(0, _HD)],
                isem.at[slot, 0]),
            pltpu.make_async_copy(
                xs_ref.at[pl.ds(_PADDED + r, _GTILE), pl.ds(0, _HD)],
                xbuf.at[slot, pl.ds(0, _GTILE), pl.ds(_HD, _HD)],
                isem.at[slot, 1]),
        )

    def out_copies(i, slot):
        r = (s0 + i) * _GTILE
        return (
            pltpu.make_async_copy(
                ybuf.at[slot, pl.ds(0, _GTILE), pl.ds(0, _HD)],
                ys_ref.at[pl.ds(r, _GTILE), pl.ds(0, _HD)],
                osem.at[slot, 0]),
            pltpu.make_async_copy(
                ybuf.at[slot, pl.ds(0, _GTILE), pl.ds(_HD, _HD)],
                ys_ref.at[pl.ds(_PADDED + r, _GTILE), pl.ds(0, _HD)],
                osem.at[slot, 1]),
        )

    def start(copies):
        for c in copies:
            c.start()

    def wait(copies):
        for c in copies:
            c.wait()

    @pl.when(n >= 1)
    def _():
        start(in_copies(0, 0))

    @pl.when(n >= 2)
    def _():
        start(in_copies(1, 1))

    def body(i, _):
        slot = lax.rem(i, 2)
        wait(in_copies(i, slot))
        xt = xbuf[slot].astype(jnp.bfloat16)
        h = lax.dot_general(xt, wfc, (((1,), (1,)), ((), ())),
                            preferred_element_type=jnp.float32)
        h = jnp.square(jnp.maximum(h, 0.0)).astype(jnp.bfloat16)
        y = lax.dot_general(h, wp, (((1,), (1,)), ((), ())),
                            preferred_element_type=jnp.float32)

        @pl.when(i >= 2)
        def _():
            wait(out_copies(i - 2, slot))

        ybuf[slot] = y
        start(out_copies(i, slot))

        @pl.when(i + 2 < n)
        def _():
            start(in_copies(i + 2, slot))

        return 0

    lax.fori_loop(0, n, body, 0)

    @pl.when(n >= 1)
    def _():
        wait(out_copies(n - 1, lax.rem(n - 1, 2)))

    @pl.when(n >= 2)
    def _():
        wait(out_copies(n - 2, lax.rem(n - 2, 2)))


def _grouped_matmul(es, ec, xs, w_fc, w_proj):
    grid_spec = pltpu.PrefetchScalarGridSpec(
        num_scalar_prefetch=2,
        grid=(_E,),
        in_specs=[
            pl.BlockSpec(memory_space=pltpu.MemorySpace.HBM),
            pl.BlockSpec((1, _H, _D), lambda e, es_r, ec_r: (e, 0, 0)),
            pl.BlockSpec((1, _D, _H), lambda e, es_r, ec_r: (e, 0, 0)),
        ],
        out_specs=pl.BlockSpec(memory_space=pltpu.MemorySpace.HBM),
        scratch_shapes=[
            pltpu.VMEM((2, _GTILE, _D), jnp.float32),
            pltpu.VMEM((2, _GTILE, _D), jnp.float32),
            pltpu.SemaphoreType.DMA((2, 2)),
            pltpu.SemaphoreType.DMA((2, 2)),
        ],
    )
    return pl.pallas_call(
        _expert_kernel,
        grid_spec=grid_spec,
        out_shape=jax.ShapeDtypeStruct((2 * _PADDED, _HD), jnp.float32),
    )(es, ec, xs, w_fc, w_proj)


def _combine_kernel(g_ref, w0_ref, w1_ref, out_ref):
    w0 = w0_ref[...]
    w1 = w1_ref[...]
    out_ref[:, 0:_HD] = (w0 * g_ref[0:_N, :] +
                         w1 * g_ref[2 * _N:3 * _N, :])
    out_ref[:, _HD:_D] = (w0 * g_ref[_N:2 * _N, :] +
                          w1 * g_ref[3 * _N:4 * _N, :])


def _combine(g, w0, w1):
    return pl.pallas_call(
        _combine_kernel,
        out_shape=jax.ShapeDtypeStruct((_N, _D), jnp.float32),
    )(g, w0, w1)


def kernel(x, gate_w, w_fc, w_proj):
    x_flat = x.reshape(_N, _D)
    x2, d1, d2, w0, w1, es, ec, loss = _route(x_flat, gate_w)
    dr1 = d1.reshape(1, _N)
    dr2 = d2.reshape(1, _N)
    dests = jnp.concatenate(
        [dr1, dr1 + _PADDED, dr2, dr2 + _PADDED], axis=1)
    xs = _sc_scatter(x2, dests)
    ys = _grouped_matmul(es, ec, xs, w_fc, w_proj)
    g = _sc_gather(ys, dests)
    out = _combine(g, w0, w1)
    return out.reshape(_B, _T, _D), loss[0, 0]


# speedup vs baseline: 1.5553x; 1.0076x over previous
import jax
import jax.numpy as jnp
from jax import lax
from jax.experimental import pallas as pl
from jax.experimental.pallas import tpu as pltpu
from jax.experimental.pallas import tpu_sc as plsc

_B, _T, _D, _E, _H = 1, 2048, 768, 8, 1536
_N = _B * _T
_HD = _D // 2
_TILE = 128
_NCHUNK = _N // _TILE
_GTILE = 256
_MAXTILES = 2 * _N // _GTILE + _E
_PADDED = _MAXTILES * _GTILE
_SCWIN = 128


def _route_kernel(x_ref, gw_ref, x2_ref, d1_ref, d2_ref, w0_ref, w1_ref,
                  es_ref, ec_ref, loss_ref):
    x = x_ref[...]
    x2_ref[0:_N, :] = x[:, 0:_HD]
    x2_ref[_N:2 * _N, :] = x[:, _HD:_D]
    gw = gw_ref[...]
    logits = lax.dot_general(
        x, gw, (((1,), (1,)), ((), ())),
        preferred_element_type=jnp.float32)
    p = jax.nn.softmax(logits, axis=-1)
    rank = jnp.zeros_like(p)
    eidx = lax.broadcasted_iota(jnp.int32, (1, _E), 1)
    for j in range(_E):
        pj = p[:, j:j + 1]
        rank += (pj > p).astype(p.dtype)
        rank += ((pj == p) & (j < eidx)).astype(p.dtype)
    sel = (rank < 2.0).astype(p.dtype)
    w = p * sel
    wd = w / jnp.sum(w, axis=1, keepdims=True)
    ii = lax.broadcasted_iota(jnp.int32, (_TILE, _TILE), 0)
    jj = lax.broadcasted_iota(jnp.int32, (_TILE, _TILE), 1)
    tril = (ii >= jj).astype(jnp.float32)
    carry = jnp.zeros((1, _E), jnp.float32)
    cum_rows = []
    for c in range(_NCHUNK):
        sc = lax.slice(sel, (c * _TILE, 0), ((c + 1) * _TILE, _E))
        cum_rows.append(
            lax.dot_general(tril, sc, (((1,), (0,)), ((), ())),
                            preferred_element_type=jnp.float32) + carry)
        carry = carry + jnp.sum(sc, axis=0, keepdims=True)
    cum = jnp.concatenate(cum_rows, axis=0)
    cnt = carry
    pmean = jnp.mean(p, axis=0, keepdims=True)
    loss_ref[...] = (jnp.sum(pmean * cnt) * (_E / _N)).reshape(1, 1)
    ptiles = jnp.floor((cnt + (_GTILE - 1.0)) * (1.0 / _GTILE))
    ltmask = (lax.broadcasted_iota(jnp.int32, (_E, _E), 0) <
              lax.broadcasted_iota(jnp.int32, (_E, _E), 1)).astype(jnp.float32)
    tstart = lax.dot_general(ptiles, ltmask, (((1,), (0,)), ((), ())),
                             preferred_element_type=jnp.float32)
    pstart = _GTILE * tstart
    kcnt = lax.dot_general(sel, ltmask, (((1,), (0,)), ((), ())),
                           preferred_element_type=jnp.float32)
    pos = pstart + cum - 1.0
    is0 = sel * (kcnt == 0.0)
    is1 = sel * (kcnt == 1.0)
    d1_ref[...] = jnp.sum(pos * is0, axis=1, keepdims=True).astype(jnp.int32)
    d2_ref[...] = jnp.sum(pos * is1, axis=1, keepdims=True).astype(jnp.int32)
    w0_ref[...] = jnp.sum(wd * is0, axis=1, keepdims=True)
    w1_ref[...] = jnp.sum(wd * is1, axis=1, keepdims=True)
    es_ref[...] = jnp.transpose(tstart).astype(jnp.int32)
    ec_ref[...] = jnp.transpose(ptiles).astype(jnp.int32)


def _route(x_flat, gate_w):
    return pl.pallas_call(
        _route_kernel,
        out_shape=[
            jax.ShapeDtypeStruct((2 * _N, _HD), jnp.float32),
            jax.ShapeDtypeStruct((_N, 1), jnp.int32),
            jax.ShapeDtypeStruct((_N, 1), jnp.int32),
            jax.ShapeDtypeStruct((_N, 1), jnp.float32),
            jax.ShapeDtypeStruct((_N, 1), jnp.float32),
            jax.ShapeDtypeStruct((_E, 1), jnp.int32),
            jax.ShapeDtypeStruct((_E, 1), jnp.int32),
            jax.ShapeDtypeStruct((1, 1), jnp.float32),
        ],
    )(x_flat, gate_w)


def _sc_scatter(x2, dests):
    @pl.kernel(
        out_type=jax.ShapeDtypeStruct((2 * _PADDED, _HD), jnp.float32),
        mesh=plsc.VectorSubcoreMesh(core_axis_name="core",
                                    subcore_axis_name="subcore"))
    def k(x_hbm, i_hbm, o_hbm):
        def body(x_vmem, i_vmem):
            pltpu.sync_copy(x_vmem, o_hbm.at[i_vmem.at[0]])

        pltpu.emit_pipeline(
            body,
            grid=(4 * _N // _SCWIN,),
            in_specs=[
                pl.BlockSpec((_SCWIN, _HD),
                             lambda i: (lax.rem(i, 2 * _N // _SCWIN), 0)),
                pl.BlockSpec((1, _SCWIN), lambda i: (0, i)),
            ],
            out_specs=[],
            core_axis_name=("core", "subcore"),
            dimension_semantics=(pltpu.PARALLEL,),
        )(x_hbm, i_hbm)

    return k(x2, dests)


def _sc_gather(ys, dests):
    @pl.kernel(
        out_type=jax.ShapeDtypeStruct((4 * _N, _HD), jnp.float32),
        mesh=plsc.VectorSubcoreMesh(core_axis_name="core",
                                    subcore_axis_name="subcore"))
    def k(y_hbm, i_hbm, o_hbm):
        def body(i_vmem, o_vmem):
            pltpu.sync_copy(y_hbm.at[i_vmem.at[0]], o_vmem)

        pltpu.emit_pipeline(
            body,
            grid=(4 * _N // _SCWIN,),
            in_specs=[pl.BlockSpec((1, _SCWIN), lambda i: (0, i))],
            out_specs=[pl.BlockSpec((_SCWIN, _HD), lambda i: (i, 0))],
            core_axis_name=("core", "subcore"),
            dimension_semantics=(pltpu.PARALLEL,),
        )(i_hbm, o_hbm)

    return k(ys, dests)


def _expert_kernel(es_ref, ec_ref, xs_ref, wfc_ref, wp_ref, ys_ref,
                   xbuf, ybuf, isem, osem):
    e = pl.program_id(0)
    s0 = es_ref[e, 0]
    n = ec_ref[e, 0]
    wfc = wfc_ref[0]
    wp = wp_ref[0]

    def in_copies(i, slot):
        r = (s0 + i) * _GTILE
        return (
            pltpu.make_async_copy(
                xs_ref.at[pl.ds(r, _GTILE), pl.ds(0, _HD)],
                xbuf.at[slot, pl.ds(0, _GTILE), pl.ds(0, _HD)],
                isem.at[slot, 0]),
            pltpu.make_async_copy(
                xs_ref.at[pl.ds(_PADDED + r, _GTILE), pl.ds(0, _HD)],
                xbuf.at[slot, pl.ds(0, _GTILE), pl.ds(_HD, _HD)],
                isem.at[slot, 1]),
        )

    def out_copies(i, slot):
        r = (s0 + i) * _GTILE
        return (
            pltpu.make_async_copy(
                ybuf.at[slot, pl.ds(0, _GTILE), pl.ds(0, _HD)],
                ys_ref.at[pl.ds(r, _GTILE), pl.ds(0, _HD)],
                osem.at[slot, 0]),
            pltpu.make_async_copy(
                ybuf.at[slot, pl.ds(0, _GTILE), pl.ds(_HD, _HD)],
                ys_ref.at[pl.ds(_PADDED + r, _GTILE), pl.ds(0, _HD)],
                osem.at[slot, 1]),
        )

    def start(copies):
        for c in copies:
            c.start()

    def wait(copies):
        for c in copies:
            c.wait()

    @pl.when(n >= 1)
    def _():
        start(in_copies(0, 0))

    @pl.when(n >= 2)
    def _():
        start(in_copies(1, 1))

    def body(i, _):
        slot = lax.rem(i, 2)
        wait(in_copies(i, slot))
        xt = xbuf[slot]
        h = lax.dot_general(xt, wfc, (((1,), (1,)), ((), ())),
                            preferred_element_type=jnp.float32)
        h = jnp.square(jnp.maximum(h, 0.0))
        y = lax.dot_general(h, wp, (((1,), (1,)), ((), ())),
                            preferred_element_type=jnp.float32)

        @pl.when(i >= 2)
        def _():
            wait(out_copies(i - 2, slot))

        ybuf[slot] = y
        start(out_copies(i, slot))

        @pl.when(i + 2 < n)
        def _():
            start(in_copies(i + 2, slot))

        return 0

    lax.fori_loop(0, n, body, 0)

    @pl.when(n >= 1)
    def _():
        wait(out_copies(n - 1, lax.rem(n - 1, 2)))

    @pl.when(n >= 2)
    def _():
        wait(out_copies(n - 2, lax.rem(n - 2, 2)))


def _grouped_matmul(es, ec, xs, w_fc, w_proj):
    grid_spec = pltpu.PrefetchScalarGridSpec(
        num_scalar_prefetch=2,
        grid=(_E,),
        in_specs=[
            pl.BlockSpec(memory_space=pltpu.MemorySpace.HBM),
            pl.BlockSpec((1, _H, _D), lambda e, es_r, ec_r: (e, 0, 0)),
            pl.BlockSpec((1, _D, _H), lambda e, es_r, ec_r: (e, 0, 0)),
        ],
        out_specs=pl.BlockSpec(memory_space=pltpu.MemorySpace.HBM),
        scratch_shapes=[
            pltpu.VMEM((2, _GTILE, _D), jnp.float32),
            pltpu.VMEM((2, _GTILE, _D), jnp.float32),
            pltpu.SemaphoreType.DMA((2, 2)),
            pltpu.SemaphoreType.DMA((2, 2)),
        ],
    )
    return pl.pallas_call(
        _expert_kernel,
        grid_spec=grid_spec,
        out_shape=jax.ShapeDtypeStruct((2 * _PADDED, _HD), jnp.float32),
    )(es, ec, xs, w_fc, w_proj)


def _combine_kernel(g_ref, w0_ref, w1_ref, out_ref):
    w0 = w0_ref[...]
    w1 = w1_ref[...]
    out_ref[:, 0:_HD] = (w0 * g_ref[0:_N, :] +
                         w1 * g_ref[2 * _N:3 * _N, :])
    out_ref[:, _HD:_D] = (w0 * g_ref[_N:2 * _N, :] +
                          w1 * g_ref[3 * _N:4 * _N, :])


def _combine(g, w0, w1):
    return pl.pallas_call(
        _combine_kernel,
        out_shape=jax.ShapeDtypeStruct((_N, _D), jnp.float32),
    )(g, w0, w1)


def kernel(x, gate_w, w_fc, w_proj):
    x_flat = x.reshape(_N, _D)
    x2, d1, d2, w0, w1, es, ec, loss = _route(x_flat, gate_w)
    dr1 = d1.reshape(1, _N)
    dr2 = d2.reshape(1, _N)
    dests = jnp.concatenate(
        [dr1, dr1 + _PADDED, dr2, dr2 + _PADDED], axis=1)
    xs = _sc_scatter(x2, dests)
    ys = _grouped_matmul(es, ec, xs, w_fc, w_proj)
    g = _sc_gather(ys, dests)
    out = _combine(g, w0, w1)
    return out.reshape(_B, _T, _D), loss[0, 0]
